# baseline (speedup 1.0000x reference)
import jax
import jax.numpy as jnp
from jax.experimental import pallas as pl
from jax.experimental.pallas import tpu as pltpu

N = 8192
K = 128
BI = 1024
NI = N // BI
G = N // 128
BIG = 2**30


def _finalize(fv, fj, a2, dist_ref, idx_ref):
    bmin = jnp.min(fv, axis=1, keepdims=True)
    lane = jax.lax.broadcasted_iota(jnp.int32, (BI, 128), 1)
    jfull = fj * 128 + lane
    idx_ref[...] = jnp.min(jnp.where(fv == bmin, jfull, BIG),
                           axis=1, keepdims=True)
    dist_ref[...] = jnp.sqrt(jnp.maximum(a2 + bmin, 0.0))


def _matcher_kernel(a_ref, b_ref, b2_ref, dist_ref, idx_ref,
                    distl_ref, idxl_ref, rv_ref, rj_ref, pa_ref):
    i = pl.program_id(0)
    ni = pl.num_programs(0)

    a = a_ref[...]
    nab = jax.lax.dot_general(
        a * -2.0, b_ref[...], (((1,), (1,)), ((), ())),
        preferred_element_type=jnp.float32)

    _finalize(rv_ref[...], rj_ref[...], pa_ref[...], dist_ref, idx_ref)

    b2 = b2_ref[...]
    val = b2[:, 0:128] + nab[:, 0:128]
    cid = jnp.zeros((BI, 128), jnp.int32)
    for g in range(1, G):
        s = b2[:, g * 128:(g + 1) * 128] + nab[:, g * 128:(g + 1) * 128]
        lt = s < val
        val = jnp.where(lt, s, val)
        cid = jnp.where(lt, jnp.int32(g), cid)
    a2 = jnp.sum(a * a, axis=1, keepdims=True)

    rv_ref[...] = val
    rj_ref[...] = cid
    pa_ref[...] = a2

    @pl.when(i == ni - 1)
    def _fin_last():
        _finalize(rv_ref[...], rj_ref[...], pa_ref[...], distl_ref, idxl_ref)


def _match(desc1, desc2, b2t):
    return pl.pallas_call(
        _matcher_kernel,
        grid=(NI,),
        in_specs=[
            pl.BlockSpec((BI, K), lambda i: (i, 0)),
            pl.BlockSpec((N, K), lambda i: (0, 0)),
            pl.BlockSpec((1, N), lambda i: (0, 0)),
        ],
        out_specs=[
            pl.BlockSpec((BI, 1), lambda i: (jnp.maximum(i - 1, 0), 0)),
            pl.BlockSpec((BI, 1), lambda i: (jnp.maximum(i - 1, 0), 0)),
            pl.BlockSpec((BI, 1), lambda i: (0, 0)),
            pl.BlockSpec((BI, 1), lambda i: (0, 0)),
        ],
        out_shape=[
            jax.ShapeDtypeStruct((N, 1), jnp.float32),
            jax.ShapeDtypeStruct((N, 1), jnp.int32),
            jax.ShapeDtypeStruct((BI, 1), jnp.float32),
            jax.ShapeDtypeStruct((BI, 1), jnp.int32),
        ],
        scratch_shapes=[
            pltpu.VMEM((BI, 128), jnp.float32),
            pltpu.VMEM((BI, 128), jnp.int32),
            pltpu.VMEM((BI, 1), jnp.float32),
        ],
    )(desc1, desc2, b2t)


def kernel(desc1, desc2):
    b2t = jnp.sum(desc2 * desc2, axis=1, keepdims=True).T
    dist_m, idx_m, dist_l, idx_l = _match(desc1, desc2, b2t)
    dists = jnp.concatenate([dist_m[:N - BI], dist_l], axis=0)
    idx2 = jnp.concatenate([idx_m[:N - BI], idx_l], axis=0)
    idx1 = jnp.arange(0, N, dtype=jnp.int32).reshape(-1, 1)
    return dists, jnp.concatenate([idx1, idx2], axis=1)


# speedup vs baseline: 1.1245x; 1.1245x over previous
import jax
import jax.numpy as jnp
from jax.experimental import pallas as pl

N = 8192
K = 128
BI = 1024
G = N // 128
BIG = 2**30


def _matcher_kernel(a_ref, b_ref, b2_ref, dist_ref, idx_ref):
    a = a_ref[...]
    nab = jax.lax.dot_general(
        a * -2.0, b_ref[...], (((1,), (1,)), ((), ())),
        preferred_element_type=jnp.float32)
    b2 = b2_ref[...]

    val = b2[:, 0:128] + nab[:, 0:128]
    cid = jnp.zeros((BI, 128), jnp.int32)
    for g in range(1, G):
        s = b2[:, g * 128:(g + 1) * 128] + nab[:, g * 128:(g + 1) * 128]
        lt = s < val
        val = jnp.where(lt, s, val)
        cid = jnp.where(lt, jnp.int32(g), cid)

    bmin = jnp.min(val, axis=1, keepdims=True)
    lane = jax.lax.broadcasted_iota(jnp.int32, (BI, 128), 1)
    jfull = cid * 128 + lane
    idx_ref[...] = jnp.min(jnp.where(val == bmin, jfull, BIG),
                           axis=1, keepdims=True)
    a2 = jnp.sum(a * a, axis=1, keepdims=True)
    dist_ref[...] = jnp.sqrt(jnp.maximum(a2 + bmin, 0.0))


def _match(desc1, desc2, b2t):
    return pl.pallas_call(
        _matcher_kernel,
        grid=(N // BI,),
        in_specs=[
            pl.BlockSpec((BI, K), lambda i: (i, 0)),
            pl.BlockSpec((N, K), lambda i: (0, 0)),
            pl.BlockSpec((1, N), lambda i: (0, 0)),
        ],
        out_specs=[
            pl.BlockSpec((BI, 1), lambda i: (i, 0)),
            pl.BlockSpec((BI, 1), lambda i: (i, 0)),
        ],
        out_shape=[
            jax.ShapeDtypeStruct((N, 1), jnp.float32),
            jax.ShapeDtypeStruct((N, 1), jnp.int32),
        ],
    )(desc1, desc2, b2t)


def kernel(desc1, desc2):
    b2t = jnp.sum(desc2 * desc2, axis=1, keepdims=True).T
    dists, idx2 = _match(desc1, desc2, b2t)
    idx1 = jnp.arange(0, N, dtype=jnp.int32).reshape(-1, 1)
    return dists, jnp.concatenate([idx1, idx2], axis=1)


# speedup vs baseline: 1.1449x; 1.0181x over previous
import jax
import jax.numpy as jnp
from jax.experimental import pallas as pl

N = 8192
K = 128
BI = 1024
G = N // 128
BIG = 2**30


def _matcher_kernel(a_ref, b_ref, b2_ref, dist_ref, idx_ref):
    a = a_ref[...]
    nab = jax.lax.dot_general(
        a * -2.0, b_ref[...], (((1,), (1,)), ((), ())),
        preferred_element_type=jnp.float32)
    b2 = b2_ref[...]

    val = b2[:, 0:128] + nab[:, 0:128]
    cid = jnp.zeros((BI, 128), jnp.int32)
    for g in range(1, G):
        s = b2[:, g * 128:(g + 1) * 128] + nab[:, g * 128:(g + 1) * 128]
        lt = s < val
        val = jnp.minimum(s, val)
        cid = jnp.where(lt, jnp.int32(g), cid)

    bmin = jnp.min(val, axis=1, keepdims=True)
    lane = jax.lax.broadcasted_iota(jnp.int32, (BI, 128), 1)
    jfull = cid * 128 + lane
    idx_ref[...] = jnp.min(jnp.where(val == bmin, jfull, BIG),
                           axis=1, keepdims=True)
    a2 = jnp.sum(a * a, axis=1, keepdims=True)
    dist_ref[...] = jnp.sqrt(jnp.maximum(a2 + bmin, 0.0))


def _match(desc1, desc2, b2t):
    return pl.pallas_call(
        _matcher_kernel,
        grid=(N // BI,),
        in_specs=[
            pl.BlockSpec((BI, K), lambda i: (i, 0)),
            pl.BlockSpec((N, K), lambda i: (0, 0)),
            pl.BlockSpec((1, N), lambda i: (0, 0)),
        ],
        out_specs=[
            pl.BlockSpec((BI, 1), lambda i: (i, 0)),
            pl.BlockSpec((BI, 1), lambda i: (i, 0)),
        ],
        out_shape=[
            jax.ShapeDtypeStruct((N, 1), jnp.float32),
            jax.ShapeDtypeStruct((N, 1), jnp.int32),
        ],
    )(desc1, desc2, b2t)


def kernel(desc1, desc2):
    b2t = jnp.sum(desc2 * desc2, axis=1, keepdims=True).T
    dists, idx2 = _match(desc1, desc2, b2t)
    idx1 = jnp.arange(0, N, dtype=jnp.int32).reshape(-1, 1)
    return dists, jnp.concatenate([idx1, idx2], axis=1)


# speedup vs baseline: 1.1469x; 1.0017x over previous
import jax
import jax.numpy as jnp
from jax.experimental import pallas as pl

N = 8192
K = 128
BI = 1024
G = N // 128
BIG = 2**30


def _matcher_kernel(a_ref, b_ref, b2_ref, dist_ref, idx_ref):
    a = a_ref[...]
    a2 = jnp.sum(a * a, axis=1, keepdims=True)
    nab = jax.lax.dot_general(
        a * -2.0, b_ref[...], (((1,), (1,)), ((), ())),
        preferred_element_type=jnp.float32)
    b2 = b2_ref[...]

    val = b2[:, 0:128] + nab[:, 0:128]
    cid = jnp.zeros((BI, 128), jnp.int32)
    for g in range(1, G):
        s = b2[:, g * 128:(g + 1) * 128] + nab[:, g * 128:(g + 1) * 128]
        lt = s < val
        val = jnp.minimum(s, val)
        cid = jnp.where(lt, jnp.int32(g), cid)

    bmin = jnp.min(val, axis=1, keepdims=True)
    lane = jax.lax.broadcasted_iota(jnp.int32, (BI, 128), 1)
    jfull = cid * 128 + lane
    idx_ref[...] = jnp.min(jnp.where(val == bmin, jfull, BIG),
                           axis=1, keepdims=True)
    dist_ref[...] = jnp.sqrt(jnp.maximum(a2 + bmin, 0.0))


def _match(desc1, desc2, b2t):
    return pl.pallas_call(
        _matcher_kernel,
        grid=(N // BI,),
        in_specs=[
            pl.BlockSpec((BI, K), lambda i: (i, 0)),
            pl.BlockSpec((N, K), lambda i: (0, 0)),
            pl.BlockSpec((1, N), lambda i: (0, 0)),
        ],
        out_specs=[
            pl.BlockSpec((BI, 1), lambda i: (i, 0)),
            pl.BlockSpec((BI, 1), lambda i: (i, 0)),
        ],
        out_shape=[
            jax.ShapeDtypeStruct((N, 1), jnp.float32),
            jax.ShapeDtypeStruct((N, 1), jnp.int32),
        ],
    )(desc1, desc2, b2t)


def kernel(desc1, desc2):
    b2t = jnp.sum(desc2 * desc2, axis=1, keepdims=True).T
    dists, idx2 = _match(desc1, desc2, b2t)
    idx1 = jnp.arange(0, N, dtype=jnp.int32).reshape(-1, 1)
    return dists, jnp.concatenate([idx1, idx2], axis=1)
